# baseline (speedup 1.0000x reference)
import functools

import jax
import jax.numpy as jnp
from jax import lax
from jax.experimental import pallas as pl
from jax.experimental.pallas import tpu as pltpu
from jax.experimental.pallas import tpu_sc as plsc

_B = 16384
_D = 64
_NW = 32
_BPW = _B // _NW
_CH = 128
_NCH = _BPW // _CH


def _make_kernel():
    mesh = plsc.VectorSubcoreMesh(core_axis_name="c", subcore_axis_name="s")

    @functools.partial(
        pl.kernel,
        mesh=mesh,
        out_type=jax.ShapeDtypeStruct((_B,), jnp.float32),
        compiler_params=pltpu.CompilerParams(use_tc_tiling_on_sc=False),
        scratch_types=[
            pltpu.VMEM((_BPW,), jnp.int32),
            pltpu.VMEM((_BPW,), jnp.int32),
            pltpu.VMEM((_BPW,), jnp.int32),
            pltpu.VMEM((_BPW, _D), jnp.float32),
            pltpu.VMEM((_BPW, _D), jnp.float32),
            pltpu.VMEM((_BPW, _D), jnp.float32),
            pltpu.VMEM((_BPW,), jnp.float32),
            pltpu.VMEM((16, 48), jnp.float32),
            pltpu.SemaphoreType.DMA,
        ],
    )
    def kge_score(s_hbm, p_hbm, o_hbm, ent_hbm, rel_hbm, out_hbm,
                  s_idx, p_idx, o_idx, se_v, pe_v, oe_v, out_v, t_v, sem):
        wid = lax.axis_index("s") * 2 + lax.axis_index("c")
        base = wid * _BPW

        pltpu.sync_copy(s_hbm.at[pl.ds(base, _BPW)], s_idx)
        pltpu.sync_copy(p_hbm.at[pl.ds(base, _BPW)], p_idx)
        pltpu.sync_copy(o_hbm.at[pl.ds(base, _BPW)], o_idx)

        cps = []
        for c in range(_NCH):
            sl = pl.ds(c * _CH, _CH)
            cps.append(pltpu.async_copy(ent_hbm.at[s_idx.at[sl]], se_v.at[sl], sem))
            cps.append(pltpu.async_copy(rel_hbm.at[p_idx.at[sl]], pe_v.at[sl], sem))
            cps.append(pltpu.async_copy(ent_hbm.at[o_idx.at[sl]], oe_v.at[sl], sem))
        for cp in cps:
            cp.wait()

        lane = lax.iota(jnp.int32, 16)

        def group(g, carry):
            acc = jnp.zeros((16,), jnp.float32)
            for j in range(16):
                b = g * 16 + j
                se0 = se_v[b, pl.ds(0, 16)]
                se1 = se_v[b, pl.ds(16, 16)]
                se2 = se_v[b, pl.ds(32, 16)]
                se3 = se_v[b, pl.ds(48, 16)]
                pe0 = pe_v[b, pl.ds(0, 16)]
                pe1 = pe_v[b, pl.ds(16, 16)]
                pe2 = pe_v[b, pl.ds(32, 16)]
                pe3 = pe_v[b, pl.ds(48, 16)]
                oe0 = oe_v[b, pl.ds(0, 16)]
                oe1 = oe_v[b, pl.ds(16, 16)]
                oe2 = oe_v[b, pl.ds(32, 16)]
                oe3 = oe_v[b, pl.ds(48, 16)]
                sp_re0 = se0 * pe0 - se2 * pe2
                sp_re1 = se1 * pe1 - se3 * pe3
                sp_im0 = se0 * pe2 + se2 * pe0
                sp_im1 = se1 * pe3 + se3 * pe1
                pv = sp_re0 * oe0 + sp_re1 * oe1 + sp_im0 * oe2 + sp_im1 * oe3
                x = pv
                for step in (8, 4, 2, 1):
                    t_v[j, pl.ds(16, 16)] = x
                    d = step if (j & step) == 0 else -step
                    x = x + t_v[j, pl.ds(16 + d, 16)]
                acc = jnp.where(lane == j, x, acc)
            out_v[pl.ds(g * 16, 16)] = acc
            return carry

        lax.fori_loop(0, _BPW // 16, group, 0)

        pltpu.sync_copy(out_v, out_hbm.at[pl.ds(base, _BPW)])

    return kge_score


_kge_score = _make_kernel()


def kernel(s, p, o, entity_emb, relation_emb):
    s = s.astype(jnp.int32)
    p = p.astype(jnp.int32)
    o = o.astype(jnp.int32)
    out = _kge_score(s, p, o, entity_emb, relation_emb)
    return out.reshape(_B, 1)
